# baseline (speedup 1.0000x reference)
import functools

import jax
import jax.numpy as jnp
from jax import lax
from jax.experimental import pallas as pl
from jax.experimental.pallas import tpu as pltpu
from jax.experimental.pallas import tpu_sc as plsc

B = 16384
EMB = 64
H1 = 128
H2 = 64
NC = 2
NS = 16
NW = NC * NS
NHIT = 2 * B
HPW = NHIT // NW
PASS = 512
W = 512
UBIT = 1 << 20
NU = 1000000
NM = 100000
WMAXU = 7813 * 128 - W
WMAXM = 782 * 128 - W
UWIN0 = UBIT // W

_mesh = plsc.VectorSubcoreMesh(core_axis_name="c", subcore_axis_name="s")


@functools.partial(
    pl.kernel,
    mesh=_mesh,
    out_type=(
        jax.ShapeDtypeStruct((B + 8, 2 * EMB), jnp.float32),
        jax.ShapeDtypeStruct((B + 8, 2 * EMB), jnp.float32),
    ),
    scratch_types=[
        pltpu.VMEM((HPW,), jnp.int32),
        pltpu.VMEM((HPW,), jnp.int32),
        pltpu.VMEM((EMB, W), jnp.float32),
        pltpu.VMEM((PASS, 2 * EMB), jnp.float32),
        pltpu.VMEM((PASS // 128, 128), jnp.int32),
        pltpu.VMEM((PASS // 128, 128), jnp.int32),
        pltpu.SemaphoreType.DMA,
        pltpu.SemaphoreType.DMA,
    ],
    compiler_params=pltpu.CompilerParams(needs_layout_passes=False),
)
def _sc_gather(keys_hbm, pos_hbm, utabT_hbm, mtabT_hbm, ue_hbm, me_hbm,
               kbuf, pbuf, wbuf, obuf, plu, plm, sem, ssem):
    wid = lax.axis_index("s") * NC + lax.axis_index("c")
    base = wid * HPW
    pltpu.sync_copy(keys_hbm.at[pl.ds(base, HPW)], kbuf)
    pltpu.sync_copy(pos_hbm.at[pl.ds(base, HPW)], pbuf)
    lanes = lax.iota(jnp.int32, 16)
    cvec = [lanes + q * 16 for q in range(EMB // 16)]
    zeros16 = jnp.zeros((16,), jnp.int32)

    def do_pass(p):
        h0 = p * PASS

        def chunk(c, carry):
            cur_w, w0c = carry
            kvec = kbuf[pl.ds(h0 + c * 16, 16)]
            for k in range(16):
                key = kvec[k]
                w = lax.shift_right_logical(key, 9)
                is_u = key >= UBIT
                r = key - jnp.where(is_u, UBIT, 0)
                w0 = jnp.minimum(w * W - jnp.where(is_u, UWIN0 * W, 0),
                                 jnp.where(is_u, WMAXU, WMAXM)).astype(jnp.int32)
                w0 = pl.multiple_of(w0, 128)
                need = w != cur_w

                @pl.when(need & is_u)
                def _():
                    pltpu.sync_copy(utabT_hbm.at[:, pl.ds(w0, W)], wbuf)

                @pl.when(need & jnp.logical_not(is_u))
                def _():
                    pltpu.sync_copy(mtabT_hbm.at[:, pl.ds(w0, W)], wbuf)

                cur_w = jnp.where(need, w, cur_w)
                w0c = jnp.where(need, w0, w0c)
                dr = r - w0c
                slot = c * 16 + k
                drv = zeros16 + dr
                for q in range(EMB // 16):
                    vals = plsc.load_gather(wbuf, [cvec[q], drv])
                    obuf[slot, pl.ds(q * 16, 16)] = vals
            return cur_w, w0c

        cur_w, w0c = lax.fori_loop(0, PASS // 16, chunk, (jnp.int32(-1), jnp.int32(0)))

        for c in range(PASS // 16):
            kvec = kbuf[pl.ds(h0 + c * 16, 16)]
            svec = pbuf[pl.ds(h0 + c * 16, 16)]
            is_u = kvec >= UBIT
            bpos = svec - jnp.where(is_u, B, 0)
            pu = jnp.where(is_u, bpos, B)
            pm = jnp.where(is_u, B, bpos)
            plu[c // 8, pl.ds((c % 8) * 16, 16)] = pu
            plm[c // 8, pl.ds((c % 8) * 16, 16)] = pm

        copies = []
        for j in range(PASS // 128):
            copies.append(pltpu.async_copy(
                obuf.at[pl.ds(j * 128, 128)], ue_hbm.at[plu.at[j]], ssem))
            copies.append(pltpu.async_copy(
                obuf.at[pl.ds(j * 128, 128)], me_hbm.at[plm.at[j]], ssem))
        for cp in copies:
            cp.wait()

    do_pass(0)
    do_pass(1)


TILE = 2048
GRID = B // TILE


def _mlp_body(ue, me, w1u, w1m, b1, w2, b2, w3, b3, out):
    h = jnp.dot(ue[:, :EMB], w1u[...], preferred_element_type=jnp.float32)
    h = h + jnp.dot(me[:, :EMB], w1m[...], preferred_element_type=jnp.float32)
    h = jnp.maximum(h + b1[...], 0.0)
    h = jnp.maximum(jnp.dot(h, w2[...], preferred_element_type=jnp.float32) + b2[...], 0.0)
    o = jnp.dot(h, w3[...], preferred_element_type=jnp.float32) + b3[...]
    out[...] = o[:, 0]


_mlp = pl.pallas_call(
    _mlp_body,
    grid=(GRID,),
    in_specs=[
        pl.BlockSpec((TILE, 2 * EMB), lambda i: (i, 0)),
        pl.BlockSpec((TILE, 2 * EMB), lambda i: (i, 0)),
        pl.BlockSpec((EMB, H1), lambda i: (0, 0)),
        pl.BlockSpec((EMB, H1), lambda i: (0, 0)),
        pl.BlockSpec((1, H1), lambda i: (0, 0)),
        pl.BlockSpec((H1, H2), lambda i: (0, 0)),
        pl.BlockSpec((1, H2), lambda i: (0, 0)),
        pl.BlockSpec((H2, 1), lambda i: (0, 0)),
        pl.BlockSpec((1, 1), lambda i: (0, 0)),
    ],
    out_specs=pl.BlockSpec((TILE,), lambda i: (i,)),
    out_shape=jax.ShapeDtypeStruct((B,), jnp.float32),
)


def kernel(user_idx, movie_idx, user_table, movie_table, W1, b1, W2, b2, W3, b3):
    uidx = user_idx.astype(jnp.int32)
    midx = movie_idx.astype(jnp.int32)
    keys = jnp.concatenate([midx, uidx + UBIT])
    vals = jnp.arange(NHIT, dtype=jnp.int32)
    sk, sv = lax.sort_key_val(keys, vals)
    ue, me = _sc_gather(sk, sv, user_table.T, movie_table.T)
    return _mlp(ue, me, W1[:EMB], W1[EMB:], b1.reshape(1, H1),
                W2, b2.reshape(1, H2), W3, b3.reshape(1, 1))


# speedup vs baseline: 1.0059x; 1.0059x over previous
import functools

import jax
import jax.numpy as jnp
from jax import lax
from jax.experimental import pallas as pl
from jax.experimental.pallas import tpu as pltpu
from jax.experimental.pallas import tpu_sc as plsc

B = 16384
EMB = 64
H1 = 128
H2 = 64
NC = 2
NS = 16
NW = NC * NS
NHIT = 2 * B
HPW = NHIT // NW
PASS = 512
W = 512
UBIT = 1 << 20
NU = 1000000
NM = 100000
WMAXU = 7813 * 128 - W
WMAXM = 782 * 128 - W
UWIN0 = UBIT // W

_mesh = plsc.VectorSubcoreMesh(core_axis_name="c", subcore_axis_name="s")


@functools.partial(
    pl.kernel,
    mesh=_mesh,
    out_type=(
        jax.ShapeDtypeStruct((B + 8, 2 * EMB), jnp.float32),
        jax.ShapeDtypeStruct((B + 8, 2 * EMB), jnp.float32),
    ),
    scratch_types=[
        pltpu.VMEM((HPW,), jnp.int32),
        pltpu.VMEM((HPW,), jnp.int32),
        pltpu.VMEM((EMB, W), jnp.float32),
        pltpu.VMEM((PASS, 2 * EMB), jnp.float32),
        pltpu.VMEM((PASS // 128, 128), jnp.int32),
        pltpu.VMEM((PASS // 128, 128), jnp.int32),
        pltpu.SemaphoreType.DMA,
        pltpu.SemaphoreType.DMA,
    ],
    compiler_params=pltpu.CompilerParams(needs_layout_passes=False),
)
def _sc_gather(keys_hbm, pos_hbm, utabT_hbm, mtabT_hbm, ue_hbm, me_hbm,
               kbuf, pbuf, wbuf, obuf, plu, plm, sem, ssem):
    wid = lax.axis_index("s") * NC + lax.axis_index("c")
    base = wid * HPW
    pltpu.sync_copy(keys_hbm.at[pl.ds(base, HPW)], kbuf)
    pltpu.sync_copy(pos_hbm.at[pl.ds(base, HPW)], pbuf)
    lanes = lax.iota(jnp.int32, 16)
    zeros16 = jnp.zeros((16,), jnp.int32)

    def do_pass(p):
        h0 = p * PASS

        def chunk(c, carry):
            cur_w0 = carry
            kvec = kbuf[pl.ds(h0 + c * 16, 16)]
            wvec = lax.shift_right_logical(kvec, 9)
            is_uv = kvec >= UBIT
            rvec = kvec - jnp.where(is_uv, UBIT, 0)
            rows = zeros16 + c * 16 + lanes

            def win_body(state):
                done, cur_w0 = state
                wsel = jnp.where(done, jnp.int32(0x7FFFFFFF), wvec)
                w_star = jnp.min(wsel)
                is_u = w_star >= UWIN0
                w0 = jnp.minimum(
                    w_star * W - jnp.where(is_u, UBIT, 0),
                    jnp.where(is_u, WMAXU, WMAXM)).astype(jnp.int32)
                w0 = pl.multiple_of(w0, 128)
                gw0 = jnp.where(is_u, w0 + NU, w0)
                need = gw0 != cur_w0

                @pl.when(need & is_u)
                def _():
                    pltpu.sync_copy(utabT_hbm.at[:, pl.ds(w0, W)], wbuf)

                @pl.when(need & jnp.logical_not(is_u))
                def _():
                    pltpu.sync_copy(mtabT_hbm.at[:, pl.ds(w0, W)], wbuf)

                m = jnp.logical_and(wvec == w_star, jnp.logical_not(done))
                drv = jnp.where(m, rvec - w0, 0)
                for cc in range(EMB):
                    vals = plsc.load_gather(wbuf, [zeros16 + cc, drv], mask=m)
                    plsc.store_scatter(obuf, [rows, zeros16 + cc], vals, mask=m)
                return jnp.logical_or(done, m), gw0

            def win_cond(state):
                done, _ = state
                return jnp.logical_not(jnp.all(done))

            done0 = jnp.zeros((16,), jnp.bool_)
            _, cur_w0 = lax.while_loop(win_cond, win_body, (done0, cur_w0))
            return cur_w0

        lax.fori_loop(0, PASS // 16, chunk, jnp.int32(-1))

        for c in range(PASS // 16):
            kvec = kbuf[pl.ds(h0 + c * 16, 16)]
            svec = pbuf[pl.ds(h0 + c * 16, 16)]
            is_u = kvec >= UBIT
            bpos = svec - jnp.where(is_u, B, 0)
            pu = jnp.where(is_u, bpos, B)
            pm = jnp.where(is_u, B, bpos)
            plu[c // 8, pl.ds((c % 8) * 16, 16)] = pu
            plm[c // 8, pl.ds((c % 8) * 16, 16)] = pm

        copies = []
        for j in range(PASS // 128):
            copies.append(pltpu.async_copy(
                obuf.at[pl.ds(j * 128, 128)], ue_hbm.at[plu.at[j]], ssem))
            copies.append(pltpu.async_copy(
                obuf.at[pl.ds(j * 128, 128)], me_hbm.at[plm.at[j]], ssem))
        for cp in copies:
            cp.wait()

    do_pass(0)
    do_pass(1)


TILE = 2048
GRID = B // TILE


def _mlp_body(ue, me, w1u, w1m, b1, w2, b2, w3, b3, out):
    h = jnp.dot(ue[:, :EMB], w1u[...], preferred_element_type=jnp.float32)
    h = h + jnp.dot(me[:, :EMB], w1m[...], preferred_element_type=jnp.float32)
    h = jnp.maximum(h + b1[...], 0.0)
    h = jnp.maximum(jnp.dot(h, w2[...], preferred_element_type=jnp.float32) + b2[...], 0.0)
    o = jnp.dot(h, w3[...], preferred_element_type=jnp.float32) + b3[...]
    out[...] = o[:, 0]


_mlp = pl.pallas_call(
    _mlp_body,
    grid=(GRID,),
    in_specs=[
        pl.BlockSpec((TILE, 2 * EMB), lambda i: (i, 0)),
        pl.BlockSpec((TILE, 2 * EMB), lambda i: (i, 0)),
        pl.BlockSpec((EMB, H1), lambda i: (0, 0)),
        pl.BlockSpec((EMB, H1), lambda i: (0, 0)),
        pl.BlockSpec((1, H1), lambda i: (0, 0)),
        pl.BlockSpec((H1, H2), lambda i: (0, 0)),
        pl.BlockSpec((1, H2), lambda i: (0, 0)),
        pl.BlockSpec((H2, 1), lambda i: (0, 0)),
        pl.BlockSpec((1, 1), lambda i: (0, 0)),
    ],
    out_specs=pl.BlockSpec((TILE,), lambda i: (i,)),
    out_shape=jax.ShapeDtypeStruct((B,), jnp.float32),
)


def kernel(user_idx, movie_idx, user_table, movie_table, W1, b1, W2, b2, W3, b3):
    uidx = user_idx.astype(jnp.int32)
    midx = movie_idx.astype(jnp.int32)
    keys = jnp.concatenate([midx, uidx + UBIT])
    vals = jnp.arange(NHIT, dtype=jnp.int32)
    sk, sv = lax.sort_key_val(keys, vals)
    ue, me = _sc_gather(sk, sv, user_table.T, movie_table.T)
    return _mlp(ue, me, W1[:EMB], W1[EMB:], b1.reshape(1, H1),
                W2, b2.reshape(1, H2), W3, b3.reshape(1, 1))


# speedup vs baseline: 1.3866x; 1.3784x over previous
import functools

import jax
import jax.numpy as jnp
from jax import lax
from jax.experimental import pallas as pl
from jax.experimental.pallas import tpu as pltpu
from jax.experimental.pallas import tpu_sc as plsc

B = 16384
EMB = 64
H1 = 128
H2 = 64
NC = 2
NS = 16
NW = NC * NS
BPW = B // NW
CHUNK = 128
NCHUNK = BPW // CHUNK

_mesh = plsc.VectorSubcoreMesh(core_axis_name="c", subcore_axis_name="s")


@functools.partial(
    pl.kernel,
    mesh=_mesh,
    out_type=(
        jax.ShapeDtypeStruct((B, 2 * EMB), jnp.float32),
        jax.ShapeDtypeStruct((B, 2 * EMB), jnp.float32),
    ),
    scratch_types=[
        pltpu.VMEM((NCHUNK, CHUNK), jnp.int32),
        pltpu.VMEM((NCHUNK, CHUNK), jnp.int32),
        pltpu.VMEM((NCHUNK, CHUNK), jnp.int32),
        pltpu.VMEM((NCHUNK, CHUNK), jnp.int32),
        pltpu.VMEM((2, CHUNK, 2 * EMB), jnp.float32),
        pltpu.VMEM((2, CHUNK, 2 * EMB), jnp.float32),
        pltpu.SemaphoreType.DMA,
        pltpu.SemaphoreType.DMA,
        pltpu.SemaphoreType.DMA,
        pltpu.SemaphoreType.DMA,
    ],
)
def _sc_gather(uidx_hbm, midx_hbm, utab_hbm, mtab_hbm, uout_hbm, mout_hbm,
               uidx_v, midx_v, uq_v, mq_v, ubuf, mbuf, su0, su1, sm0, sm1):
    wid = lax.axis_index("s") * NC + lax.axis_index("c")
    row0 = wid * NCHUNK
    base = wid * BPW
    pltpu.sync_copy(uidx_hbm.at[pl.ds(row0, NCHUNK)], uidx_v)
    pltpu.sync_copy(midx_hbm.at[pl.ds(row0, NCHUNK)], midx_v)
    for j in range(NCHUNK):
        for k in range(CHUNK // 16):
            o = k * 16
            uq_v[j, pl.ds(o, 16)] = lax.shift_right_logical(uidx_v[j, pl.ds(o, 16)], 1)
            mq_v[j, pl.ds(o, 16)] = lax.shift_right_logical(midx_v[j, pl.ds(o, 16)], 1)

    usems = (su0, su1)
    msems = (sm0, sm1)

    def fire(j):
        cu = pltpu.async_copy(utab_hbm.at[uq_v.at[j]], ubuf.at[j % 2], usems[j % 2])
        cm = pltpu.async_copy(mtab_hbm.at[mq_v.at[j]], mbuf.at[j % 2], msems[j % 2])
        return cu, cm

    inflight = [fire(0), fire(1)]
    for j in range(NCHUNK):
        cu, cm = inflight[j % 2]
        cu.wait()
        pltpu.sync_copy(ubuf.at[j % 2], uout_hbm.at[pl.ds(base + j * CHUNK, CHUNK)])
        cm.wait()
        pltpu.sync_copy(mbuf.at[j % 2], mout_hbm.at[pl.ds(base + j * CHUNK, CHUNK)])
        if j + 2 < NCHUNK:
            inflight[j % 2] = fire(j + 2)


TILE = 2048
GRID = B // TILE


def _mlp_body(up, mp, paru, parm, w1u, w1m, b1, w2, b2, w3, b3, out):
    ue = jnp.where(paru[...] > 0.5, up[:, EMB:], up[:, :EMB])
    me = jnp.where(parm[...] > 0.5, mp[:, EMB:], mp[:, :EMB])
    h = jnp.dot(ue, w1u[...], preferred_element_type=jnp.float32)
    h = h + jnp.dot(me, w1m[...], preferred_element_type=jnp.float32)
    h = jnp.maximum(h + b1[...], 0.0)
    h = jnp.maximum(jnp.dot(h, w2[...], preferred_element_type=jnp.float32) + b2[...], 0.0)
    o = jnp.dot(h, w3[...], preferred_element_type=jnp.float32) + b3[...]
    out[...] = o[:, 0]


_mlp = pl.pallas_call(
    _mlp_body,
    grid=(GRID,),
    in_specs=[
        pl.BlockSpec((TILE, 2 * EMB), lambda i: (i, 0)),
        pl.BlockSpec((TILE, 2 * EMB), lambda i: (i, 0)),
        pl.BlockSpec((TILE, 1), lambda i: (i, 0)),
        pl.BlockSpec((TILE, 1), lambda i: (i, 0)),
        pl.BlockSpec((EMB, H1), lambda i: (0, 0)),
        pl.BlockSpec((EMB, H1), lambda i: (0, 0)),
        pl.BlockSpec((1, H1), lambda i: (0, 0)),
        pl.BlockSpec((H1, H2), lambda i: (0, 0)),
        pl.BlockSpec((1, H2), lambda i: (0, 0)),
        pl.BlockSpec((H2, 1), lambda i: (0, 0)),
        pl.BlockSpec((1, 1), lambda i: (0, 0)),
    ],
    out_specs=pl.BlockSpec((TILE,), lambda i: (i,)),
    out_shape=jax.ShapeDtypeStruct((B,), jnp.float32),
)


def kernel(user_idx, movie_idx, user_table, movie_table, W1, b1, W2, b2, W3, b3):
    uidx = user_idx.astype(jnp.int32)
    midx = movie_idx.astype(jnp.int32)
    ut2 = user_table.reshape(user_table.shape[0] // 2, 2 * EMB)
    mt2 = movie_table.reshape(movie_table.shape[0] // 2, 2 * EMB)
    up, mp = _sc_gather(uidx.reshape(B // CHUNK, CHUNK),
                        midx.reshape(B // CHUNK, CHUNK), ut2, mt2)
    paru = (uidx & 1).astype(jnp.float32).reshape(B, 1)
    parm = (midx & 1).astype(jnp.float32).reshape(B, 1)
    return _mlp(up, mp, paru, parm, W1[:EMB], W1[EMB:], b1.reshape(1, H1),
                W2, b2.reshape(1, H2), W3, b3.reshape(1, 1))
